# baseline (speedup 1.0000x reference)
import functools

import jax
import jax.numpy as jnp
from jax import lax
from jax.experimental import pallas as pl
from jax.experimental.pallas import tpu as pltpu
from jax.experimental.pallas import tpu_sc as plsc

_B = 16384
_L = 50
_D = 32
_V = 1000000
_NW = 32
_BPW = _B // _NW
_R = 32
_NCHUNK = _BPW // _R
_NIDX = _R * _L

_G = 489
_VPAD = _G * 2048


def _tc_retile_body(wt_ref, out_ref):
    eye = jnp.eye(128, dtype=jnp.float32)
    for cc in range(4):
        blk = wt_ref[:, cc * 512:(cc + 1) * 512]
        g = jnp.concatenate(
            [blk[:, a * 128:(a + 1) * 128] for a in range(4)], axis=0)
        t = lax.dot_general(g, eye, (((0,), (0,)), ((), ())),
                            precision=lax.Precision.HIGHEST,
                            preferred_element_type=jnp.float32)
        out_ref[cc * 128:(cc + 1) * 128, :] = t


def _body(x_hbm, xlen_hbm, tab_hbm, out_hbm,
          xlen_v, idx_v, idxm_v, rows_v, out_v, sem):
    wid = lax.axis_index("s") * 2 + lax.axis_index("c")
    b0 = wid * _BPW

    pltpu.sync_copy(xlen_hbm.at[pl.ds(b0, _BPW)], xlen_v.at[pl.ds(0, _BPW)])

    z16 = jnp.zeros((16,), jnp.float32)
    c_hi = jnp.full((16,), -512, jnp.int32)
    c_127 = jnp.full((16,), 127, jnp.int32)
    c_3 = jnp.full((16,), 3, jnp.int32)
    c_s7 = jnp.full((16,), 7, jnp.int32)
    c_s2 = jnp.full((16,), 2, jnp.int32)

    def chunk_body(c, carry):
        cb = c * _R

        pltpu.sync_copy(x_hbm.at[pl.ds((b0 + cb) * _L, _NIDX)],
                        idx_v.at[pl.ds(0, _NIDX)])

        for t in range(_NIDX // 16):
            r = idx_v[pl.ds(t * 16, 16)]
            hi = r & c_hi
            s2 = lax.shift_left(r & c_127, c_s2)
            a = lax.shift_right_logical(r, c_s7) & c_3
            idxm_v[pl.ds(t * 16, 16)] = (hi | s2) | a

        descs = []
        for rr in range(12):
            descs.append(pltpu.async_copy(
                tab_hbm.at[idxm_v.at[pl.ds(rr * 128, 128)]],
                rows_v.at[pl.ds(rr * 128, 128)], sem))
        descs.append(pltpu.async_copy(
            tab_hbm.at[idxm_v.at[pl.ds(1536, 64)]],
            rows_v.at[pl.ds(1536, 64)], sem))
        for dsc in descs:
            dsc.wait()

        def row_body(i, inner_carry):
            base = i * _L
            ln = xlen_v[pl.ds(cb + i, 16)][0]

            def jbody(j, acc):
                a0, a1 = acc
                v = idx_v[pl.ds(base + j, 16)][0]
                r0 = rows_v[base + j, pl.ds(0, 16)]
                r1 = rows_v[base + j, pl.ds(16, 16)]
                nz = v != 0
                a0 = a0 + jnp.where(nz, r0, z16)
                a1 = a1 + jnp.where(nz, r1, z16)
                return (a0, a1)

            a0, a1 = lax.fori_loop(0, ln, jbody, (z16, z16))
            lnv = jnp.broadcast_to(ln, (16,))
            lnf = lnv.astype(jnp.float32)
            rv = jnp.where(lnv > 0, 1.0 / lnf, jnp.zeros((16,), jnp.float32))
            out_v[pl.ds(i * _D, 16)] = a0 * rv
            out_v[pl.ds(i * _D + 16, 16)] = a1 * rv
            return inner_carry

        lax.fori_loop(0, _R, row_body, 0)

        pltpu.sync_copy(out_v, out_hbm.at[pl.ds((b0 + cb) * _D, _R * _D)])
        return carry

    lax.fori_loop(0, _NCHUNK, chunk_body, 0)


@jax.jit
def kernel(x, x_len, embed_weight):
    x_flat = x.astype(jnp.int32).reshape(-1)

    wt = jnp.transpose(embed_weight, (1, 0))
    tab128 = pl.pallas_call(
        _tc_retile_body,
        grid=(_G,),
        in_specs=[pl.BlockSpec((_D, 2048), lambda g: (0, g))],
        out_specs=pl.BlockSpec((512, 128), lambda g: (g, 0)),
        out_shape=jax.ShapeDtypeStruct((_G * 512, 128), jnp.float32),
    )(wt)
    tab = tab128.reshape(-1).reshape(_VPAD, _D)

    mesh = plsc.VectorSubcoreMesh(core_axis_name="c", subcore_axis_name="s")
    f = functools.partial(
        pl.kernel,
        out_type=jax.ShapeDtypeStruct((_B * _D,), jnp.float32),
        mesh=mesh,
        compiler_params=pltpu.CompilerParams(use_tc_tiling_on_sc=False),
        scratch_types=[
            pltpu.VMEM((_BPW + 16,), jnp.int32),
            pltpu.VMEM((_NIDX + 16,), jnp.int32),
            pltpu.VMEM((_NIDX,), jnp.int32),
            pltpu.VMEM((_NIDX, _D), jnp.float32),
            pltpu.VMEM((_R * _D,), jnp.float32),
            pltpu.SemaphoreType.DMA,
        ],
    )(_body)
    out = f(x_flat, x_len.astype(jnp.int32), tab)
    return out.reshape(_B, _D)


# speedup vs baseline: 1.5760x; 1.5760x over previous
import functools

import jax
import jax.numpy as jnp
from jax import lax
from jax.experimental import pallas as pl
from jax.experimental.pallas import tpu as pltpu
from jax.experimental.pallas import tpu_sc as plsc

_B = 16384
_L = 50
_D = 32
_V = 1000000
_NW = 32
_BPW = _B // _NW
_R = 32
_NCHUNK = _BPW // _R
_NIDX = _R * _L

_G = 123
_VPAD = _G * 8192


def _tc_retile_body(wt_ref, out_ref):
    eye = jnp.eye(128, dtype=jnp.float32)
    for cc in range(16):
        blk = wt_ref[:, cc * 512:(cc + 1) * 512]
        g = jnp.concatenate(
            [blk[:, a * 128:(a + 1) * 128] for a in range(4)], axis=0)
        t = lax.dot_general(g, eye, (((0,), (0,)), ((), ())),
                            precision=lax.Precision.HIGHEST,
                            preferred_element_type=jnp.float32)
        out_ref[cc * 128:(cc + 1) * 128, :] = t


def _body(x_hbm, xlen_hbm, tab_hbm, out_hbm,
          xlen_v, idx_v, idxm_v, rows_v, out_v, sem):
    wid = lax.axis_index("s") * 2 + lax.axis_index("c")
    b0 = wid * _BPW

    pltpu.sync_copy(xlen_hbm.at[pl.ds(b0, _BPW)], xlen_v.at[pl.ds(0, _BPW)])

    z16 = jnp.zeros((16,), jnp.float32)
    c_hi = jnp.full((16,), -512, jnp.int32)
    c_127 = jnp.full((16,), 127, jnp.int32)
    c_3 = jnp.full((16,), 3, jnp.int32)
    c_s7 = jnp.full((16,), 7, jnp.int32)
    c_s2 = jnp.full((16,), 2, jnp.int32)

    def chunk_body(c, carry):
        cb = c * _R

        pltpu.sync_copy(x_hbm.at[pl.ds((b0 + cb) * _L, _NIDX)],
                        idx_v.at[pl.ds(0, _NIDX)])

        for t in range(_NIDX // 16):
            r = idx_v[pl.ds(t * 16, 16)]
            hi = r & c_hi
            s2 = lax.shift_left(r & c_127, c_s2)
            a = lax.shift_right_logical(r, c_s7) & c_3
            idxm_v[pl.ds(t * 16, 16)] = (hi | s2) | a

        descs = []
        for rr in range(12):
            descs.append(pltpu.async_copy(
                tab_hbm.at[idxm_v.at[pl.ds(rr * 128, 128)]],
                rows_v.at[pl.ds(rr * 128, 128)], sem))
        descs.append(pltpu.async_copy(
            tab_hbm.at[idxm_v.at[pl.ds(1536, 64)]],
            rows_v.at[pl.ds(1536, 64)], sem))
        for dsc in descs:
            dsc.wait()

        def row_body(i, inner_carry):
            base = i * _L
            ln = xlen_v[pl.ds(cb + i, 16)][0]

            def jbody(j, acc):
                a0, a1 = acc
                v = idx_v[pl.ds(base + j, 16)][0]
                r0 = rows_v[base + j, pl.ds(0, 16)]
                r1 = rows_v[base + j, pl.ds(16, 16)]
                nz = v != 0
                a0 = a0 + jnp.where(nz, r0, z16)
                a1 = a1 + jnp.where(nz, r1, z16)
                return (a0, a1)

            a0, a1 = lax.fori_loop(0, ln, jbody, (z16, z16))
            lnv = jnp.broadcast_to(ln, (16,))
            lnf = lnv.astype(jnp.float32)
            rv = jnp.where(lnv > 0, 1.0 / lnf, jnp.zeros((16,), jnp.float32))
            out_v[pl.ds(i * _D, 16)] = a0 * rv
            out_v[pl.ds(i * _D + 16, 16)] = a1 * rv
            return inner_carry

        lax.fori_loop(0, _R, row_body, 0)

        pltpu.sync_copy(out_v, out_hbm.at[pl.ds((b0 + cb) * _D, _R * _D)])
        return carry

    lax.fori_loop(0, _NCHUNK, chunk_body, 0)


@jax.jit
def kernel(x, x_len, embed_weight):
    x_flat = x.astype(jnp.int32).reshape(-1)

    wt = jnp.transpose(embed_weight, (1, 0))
    tab128 = pl.pallas_call(
        _tc_retile_body,
        grid=(_G,),
        in_specs=[pl.BlockSpec((_D, 8192), lambda g: (0, g))],
        out_specs=pl.BlockSpec((2048, 128), lambda g: (g, 0)),
        out_shape=jax.ShapeDtypeStruct((_G * 2048, 128), jnp.float32),
    )(wt)
    tab = tab128.reshape(-1).reshape(_VPAD, _D)

    mesh = plsc.VectorSubcoreMesh(core_axis_name="c", subcore_axis_name="s")
    f = functools.partial(
        pl.kernel,
        out_type=jax.ShapeDtypeStruct((_B * _D,), jnp.float32),
        mesh=mesh,
        compiler_params=pltpu.CompilerParams(use_tc_tiling_on_sc=False),
        scratch_types=[
            pltpu.VMEM((_BPW + 16,), jnp.int32),
            pltpu.VMEM((_NIDX + 16,), jnp.int32),
            pltpu.VMEM((_NIDX,), jnp.int32),
            pltpu.VMEM((_NIDX, _D), jnp.float32),
            pltpu.VMEM((_R * _D,), jnp.float32),
            pltpu.SemaphoreType.DMA,
        ],
    )(_body)
    out = f(x_flat, x_len.astype(jnp.int32), tab)
    return out.reshape(_B, _D)


# speedup vs baseline: 1.7535x; 1.1127x over previous
import functools

import jax
import jax.numpy as jnp
from jax import lax
from jax.experimental import pallas as pl
from jax.experimental.pallas import tpu as pltpu
from jax.experimental.pallas import tpu_sc as plsc

_B = 16384
_L = 50
_D = 32
_V = 1000000
_NW = 32
_BPW = _B // _NW
_R = 32
_NCHUNK = _BPW // _R
_NIDX = _R * _L

_G = 62
_VPAD = _G * 16384


def _tc_retile_body(wt_ref, out_ref):
    eye = jnp.eye(128, dtype=jnp.float32)
    for cc in range(32):
        blk = wt_ref[:, cc * 512:(cc + 1) * 512]
        g = jnp.concatenate(
            [blk[:, a * 128:(a + 1) * 128] for a in range(4)], axis=0)
        t = lax.dot_general(g, eye, (((0,), (0,)), ((), ())),
                            precision=lax.Precision.HIGHEST,
                            preferred_element_type=jnp.float32)
        out_ref[cc * 128:(cc + 1) * 128, :] = t


def _body(x_hbm, xlen_hbm, tab_hbm, out_hbm,
          xlen_v, idx_v, idxm_v, rows_v, out_v, sem):
    wid = lax.axis_index("s") * 2 + lax.axis_index("c")
    b0 = wid * _BPW

    pltpu.sync_copy(xlen_hbm.at[pl.ds(b0, _BPW)], xlen_v.at[pl.ds(0, _BPW)])

    z16 = jnp.zeros((16,), jnp.float32)
    c_hi = jnp.full((16,), -512, jnp.int32)
    c_127 = jnp.full((16,), 127, jnp.int32)
    c_3 = jnp.full((16,), 3, jnp.int32)
    c_s7 = jnp.full((16,), 7, jnp.int32)
    c_s2 = jnp.full((16,), 2, jnp.int32)

    def chunk_body(c, carry):
        cb = c * _R

        pltpu.sync_copy(x_hbm.at[pl.ds((b0 + cb) * _L, _NIDX)],
                        idx_v.at[pl.ds(0, _NIDX)])

        for t in range(_NIDX // 16):
            r = idx_v[pl.ds(t * 16, 16)]
            hi = r & c_hi
            s2 = lax.shift_left(r & c_127, c_s2)
            a = lax.shift_right_logical(r, c_s7) & c_3
            idxm_v[pl.ds(t * 16, 16)] = (hi | s2) | a

        descs = []
        for rr in range(12):
            descs.append(pltpu.async_copy(
                tab_hbm.at[idxm_v.at[pl.ds(rr * 128, 128)]],
                rows_v.at[pl.ds(rr * 128, 128)], sem))
        descs.append(pltpu.async_copy(
            tab_hbm.at[idxm_v.at[pl.ds(1536, 64)]],
            rows_v.at[pl.ds(1536, 64)], sem))
        for dsc in descs:
            dsc.wait()

        def row_body(i, inner_carry):
            base = i * _L
            ln = xlen_v[pl.ds(cb + i, 16)][0]

            def jbody(j, acc):
                a0, a1 = acc
                v = idx_v[pl.ds(base + j, 16)][0]
                r0 = rows_v[base + j, pl.ds(0, 16)]
                r1 = rows_v[base + j, pl.ds(16, 16)]
                nz = v != 0
                a0 = a0 + jnp.where(nz, r0, z16)
                a1 = a1 + jnp.where(nz, r1, z16)
                return (a0, a1)

            a0, a1 = lax.fori_loop(0, ln, jbody, (z16, z16))
            lnv = jnp.broadcast_to(ln, (16,))
            lnf = lnv.astype(jnp.float32)
            rv = jnp.where(lnv > 0, 1.0 / lnf, jnp.zeros((16,), jnp.float32))
            out_v[pl.ds(i * _D, 16)] = a0 * rv
            out_v[pl.ds(i * _D + 16, 16)] = a1 * rv
            return inner_carry

        lax.fori_loop(0, _R, row_body, 0)

        pltpu.sync_copy(out_v, out_hbm.at[pl.ds((b0 + cb) * _D, _R * _D)])
        return carry

    lax.fori_loop(0, _NCHUNK, chunk_body, 0)


@jax.jit
def kernel(x, x_len, embed_weight):
    x_flat = x.astype(jnp.int32).reshape(-1)

    wt = jnp.transpose(embed_weight, (1, 0))
    tab128 = pl.pallas_call(
        _tc_retile_body,
        grid=(_G,),
        in_specs=[pl.BlockSpec((_D, 16384), lambda g: (0, g))],
        out_specs=pl.BlockSpec((4096, 128), lambda g: (g, 0)),
        out_shape=jax.ShapeDtypeStruct((_G * 4096, 128), jnp.float32),
    )(wt)
    tab = tab128.reshape(-1).reshape(_VPAD, _D)

    mesh = plsc.VectorSubcoreMesh(core_axis_name="c", subcore_axis_name="s")
    f = functools.partial(
        pl.kernel,
        out_type=jax.ShapeDtypeStruct((_B * _D,), jnp.float32),
        mesh=mesh,
        compiler_params=pltpu.CompilerParams(use_tc_tiling_on_sc=False),
        scratch_types=[
            pltpu.VMEM((_BPW + 16,), jnp.int32),
            pltpu.VMEM((_NIDX + 16,), jnp.int32),
            pltpu.VMEM((_NIDX,), jnp.int32),
            pltpu.VMEM((_NIDX, _D), jnp.float32),
            pltpu.VMEM((_R * _D,), jnp.float32),
            pltpu.SemaphoreType.DMA,
        ],
    )(_body)
    out = f(x_flat, x_len.astype(jnp.int32), tab)
    return out.reshape(_B, _D)


# speedup vs baseline: 1.8573x; 1.0592x over previous
import functools

import jax
import jax.numpy as jnp
from jax import lax
from jax.experimental import pallas as pl
from jax.experimental.pallas import tpu as pltpu
from jax.experimental.pallas import tpu_sc as plsc

_B = 16384
_L = 50
_D = 32
_V = 1000000
_NW = 32
_BPW = _B // _NW
_R = 32
_NCHUNK = _BPW // _R
_NIDX = _R * _L

_G = 31
_VPAD = _G * 32768


def _tc_retile_body(wt_ref, out_ref):
    eye = jnp.eye(128, dtype=jnp.float32)
    for cc in range(64):
        blk = wt_ref[:, cc * 512:(cc + 1) * 512]
        g = jnp.concatenate(
            [blk[:, a * 128:(a + 1) * 128] for a in range(4)], axis=0)
        t = lax.dot_general(g, eye, (((0,), (0,)), ((), ())),
                            precision=lax.Precision.HIGHEST,
                            preferred_element_type=jnp.float32)
        out_ref[cc * 128:(cc + 1) * 128, :] = t


def _body(x_hbm, xlen_hbm, tab_hbm, out_hbm,
          xlen_v, idx_v, idxm_v, rows_v, out_v, sem):
    wid = lax.axis_index("s") * 2 + lax.axis_index("c")
    b0 = wid * _BPW

    pltpu.sync_copy(xlen_hbm.at[pl.ds(b0, _BPW)], xlen_v.at[pl.ds(0, _BPW)])

    z16 = jnp.zeros((16,), jnp.float32)
    c_hi = jnp.full((16,), -512, jnp.int32)
    c_127 = jnp.full((16,), 127, jnp.int32)
    c_3 = jnp.full((16,), 3, jnp.int32)
    c_s7 = jnp.full((16,), 7, jnp.int32)
    c_s2 = jnp.full((16,), 2, jnp.int32)

    def chunk_body(c, carry):
        cb = c * _R

        pltpu.sync_copy(x_hbm.at[pl.ds((b0 + cb) * _L, _NIDX)],
                        idx_v.at[pl.ds(0, _NIDX)])

        for t in range(_NIDX // 16):
            r = idx_v[pl.ds(t * 16, 16)]
            hi = r & c_hi
            s2 = lax.shift_left(r & c_127, c_s2)
            a = lax.shift_right_logical(r, c_s7) & c_3
            idxm_v[pl.ds(t * 16, 16)] = (hi | s2) | a

        descs = []
        for rr in range(12):
            descs.append(pltpu.async_copy(
                tab_hbm.at[idxm_v.at[pl.ds(rr * 128, 128)]],
                rows_v.at[pl.ds(rr * 128, 128)], sem))
        descs.append(pltpu.async_copy(
            tab_hbm.at[idxm_v.at[pl.ds(1536, 64)]],
            rows_v.at[pl.ds(1536, 64)], sem))
        for dsc in descs:
            dsc.wait()

        def row_body(i, inner_carry):
            base = i * _L
            ln = xlen_v[pl.ds(cb + i, 16)][0]

            def jbody(j, acc):
                a0, a1 = acc
                v = idx_v[pl.ds(base + j, 16)][0]
                r0 = rows_v[base + j, pl.ds(0, 16)]
                r1 = rows_v[base + j, pl.ds(16, 16)]
                nz = v != 0
                a0 = a0 + jnp.where(nz, r0, z16)
                a1 = a1 + jnp.where(nz, r1, z16)
                return (a0, a1)

            a0, a1 = lax.fori_loop(0, ln, jbody, (z16, z16))
            lnv = jnp.broadcast_to(ln, (16,))
            lnf = lnv.astype(jnp.float32)
            rv = jnp.where(lnv > 0, 1.0 / lnf, jnp.zeros((16,), jnp.float32))
            out_v[pl.ds(i * _D, 16)] = a0 * rv
            out_v[pl.ds(i * _D + 16, 16)] = a1 * rv
            return inner_carry

        lax.fori_loop(0, _R, row_body, 0)

        pltpu.sync_copy(out_v, out_hbm.at[pl.ds((b0 + cb) * _D, _R * _D)])
        return carry

    lax.fori_loop(0, _NCHUNK, chunk_body, 0)


@jax.jit
def kernel(x, x_len, embed_weight):
    x_flat = x.astype(jnp.int32).reshape(-1)

    wt = jnp.transpose(embed_weight, (1, 0))
    tab128 = pl.pallas_call(
        _tc_retile_body,
        grid=(_G,),
        in_specs=[pl.BlockSpec((_D, 32768), lambda g: (0, g))],
        out_specs=pl.BlockSpec((8192, 128), lambda g: (g, 0)),
        out_shape=jax.ShapeDtypeStruct((_G * 8192, 128), jnp.float32),
    )(wt)
    tab = tab128.reshape(-1).reshape(_VPAD, _D)

    mesh = plsc.VectorSubcoreMesh(core_axis_name="c", subcore_axis_name="s")
    f = functools.partial(
        pl.kernel,
        out_type=jax.ShapeDtypeStruct((_B * _D,), jnp.float32),
        mesh=mesh,
        compiler_params=pltpu.CompilerParams(use_tc_tiling_on_sc=False),
        scratch_types=[
            pltpu.VMEM((_BPW + 16,), jnp.int32),
            pltpu.VMEM((_NIDX + 16,), jnp.int32),
            pltpu.VMEM((_NIDX,), jnp.int32),
            pltpu.VMEM((_NIDX, _D), jnp.float32),
            pltpu.VMEM((_R * _D,), jnp.float32),
            pltpu.SemaphoreType.DMA,
        ],
    )(_body)
    out = f(x_flat, x_len.astype(jnp.int32), tab)
    return out.reshape(_B, _D)
